# Initial kernel scaffold; baseline (speedup 1.0000x reference)
import jax
import jax.numpy as jnp
from jax.experimental import pallas as pl


def kernel(xyz, edge_index, cell):
    raise NotImplementedError("write your pallas kernel here")


# speedup vs baseline: 8.1571x; 8.1571x over previous
import functools

import jax
import jax.numpy as jnp
from jax import lax
from jax.experimental import pallas as pl
from jax.experimental.pallas import tpu as pltpu
from jax.experimental.pallas import tpu_sc as plsc

_NC = 2
_NS = 16
_NW = _NC * _NS
_L = 16

_K = 2000


def _newton_rsqrt(x):
    i = plsc.bitcast(x, jnp.int32)
    i = jnp.int32(0x5F3759DF) - (i >> 1)
    y = plsc.bitcast(i, jnp.float32)
    xh = x * 0.5
    for _ in range(3):
        y = y * (1.5 - xh * y * y)
    return y


def _build(E):
    per_w = E // _NW
    n_chunks = per_w // _K
    assert per_w % _K == 0 and _K % _L == 0
    mesh = plsc.VectorSubcoreMesh(core_axis_name="c", subcore_axis_name="s",
                                  num_cores=_NC, num_subcores=_NS)

    @functools.partial(
        pl.kernel,
        mesh=mesh,
        compiler_params=pltpu.CompilerParams(needs_layout_passes=False),
        out_type=(
            jax.ShapeDtypeStruct((E,), jnp.float32),
            jax.ShapeDtypeStruct((3 * E,), jnp.float32),
        ),
        scratch_types=[
            pltpu.VMEM((3, _L), jnp.float32),
            pltpu.VMEM((_K,), jnp.int32),
            pltpu.VMEM((_K,), jnp.int32),
            [pltpu.VMEM((_K,), jnp.float32) for _ in range(3)],
            [pltpu.VMEM((_K,), jnp.float32) for _ in range(3)],
            pltpu.VMEM((_K,), jnp.float32),
            pltpu.VMEM((3 * _K,), jnp.float32),
            pltpu.SemaphoreType.DMA,
        ],
    )
    def dist_kernel(xs_hbm, ys_hbm, zs_hbm, src_hbm, dst_hbm, cell_hbm,
                    dis_hbm, out2_hbm,
                    cell_v, idx_s, idx_d, s_comp, d_comp, dis_v, out2_v,
                    sem):
        wid = lax.axis_index("s") * _NC + lax.axis_index("c")
        base = wid * per_w
        tables = (xs_hbm, ys_hbm, zs_hbm)

        pltpu.sync_copy(cell_hbm, cell_v)
        cell_c = (cell_v[0], cell_v[1], cell_v[2])
        iota3 = lax.iota(jnp.int32, _L) * 3

        def chunk_body(c, carry):
            gbase = base + c * _K
            pltpu.sync_copy(src_hbm.at[pl.ds(gbase, _K)], idx_s)
            pltpu.sync_copy(dst_hbm.at[pl.ds(gbase, _K)], idx_d)
            cps = [pltpu.async_copy(tables[k].at[idx_s], s_comp[k], sem)
                   for k in range(3)]
            cps += [pltpu.async_copy(tables[k].at[idx_d], d_comp[k], sem)
                    for k in range(3)]
            for cp in cps:
                cp.wait()

            def grp(g, carry2):
                e3 = g * (3 * _L) + iota3
                sumsq = jnp.zeros((_L,), jnp.float32)
                for comp in range(3):
                    s = s_comp[comp][pl.ds(g * _L, _L)]
                    d = d_comp[comp][pl.ds(g * _L, _L)]
                    dv = s - d
                    adv = jnp.abs(dv)
                    dv2 = jnp.minimum(cell_c[comp] - adv, adv)
                    mask2 = jnp.where(jnp.abs(dv2) == adv, 1.0, -1.0)
                    w = dv + 1e-8
                    mask = w / jnp.abs(w)
                    plsc.store_scatter(out2_v, [e3 + comp],
                                       dv2 * mask2 * mask)
                    sv = dv + 1e-9
                    sumsq = sumsq + sv * sv
                dis_v[pl.ds(g * _L, _L)] = sumsq * _newton_rsqrt(sumsq)
                return carry2

            lax.fori_loop(0, _K // _L, grp, 0)
            pltpu.sync_copy(dis_v, dis_hbm.at[pl.ds(gbase, _K)])
            pltpu.sync_copy(out2_v, out2_hbm.at[pl.ds(3 * gbase, 3 * _K)])
            return carry

        lax.fori_loop(0, n_chunks, chunk_body, 0)

    return dist_kernel


@jax.jit
def kernel(xyz, edge_index, cell):
    E = edge_index.shape[1]
    src = edge_index[0]
    dst = edge_index[1]
    xs, ys, zs = xyz[:, 0], xyz[:, 1], xyz[:, 2]
    cell16 = jnp.broadcast_to(cell.reshape(3, 1), (3, _L)).astype(jnp.float32)
    dis, out2f = _build(E)(xs, ys, zs, src, dst, cell16)
    return dis, out2f.reshape(E, 3)


# speedup vs baseline: 9.1284x; 1.1191x over previous
import functools

import jax
import jax.numpy as jnp
from jax import lax
from jax.experimental import pallas as pl
from jax.experimental.pallas import tpu as pltpu
from jax.experimental.pallas import tpu_sc as plsc

_NC = 2
_NS = 16
_NW = _NC * _NS
_L = 16

_K = 2048


def _newton_rsqrt(x):
    i = plsc.bitcast(x, jnp.int32)
    i = jnp.int32(0x5F3759DF) - (i >> 1)
    y = plsc.bitcast(i, jnp.float32)
    xh = x * 0.5
    for _ in range(3):
        y = y * (1.5 - xh * y * y)
    return y


def _build(E):
    n_chunks = E // _K
    assert E % _K == 0 and _K % _L == 0
    n_iter = -(-n_chunks // _NW)
    if n_iter % 2:
        n_iter += 1
    mesh = plsc.VectorSubcoreMesh(core_axis_name="c", subcore_axis_name="s",
                                  num_cores=_NC, num_subcores=_NS)

    @functools.partial(
        pl.kernel,
        mesh=mesh,
        compiler_params=pltpu.CompilerParams(needs_layout_passes=False),
        out_type=(
            jax.ShapeDtypeStruct((E,), jnp.float32),
            jax.ShapeDtypeStruct((3 * E,), jnp.float32),
        ),
        scratch_types=[
            pltpu.VMEM((3, _L), jnp.float32),
            [pltpu.VMEM((2, _K), jnp.int32) for _ in range(2)],
            [[pltpu.VMEM((_K,), jnp.int32) for _ in range(3)]
             for _ in range(2)],
            [[pltpu.VMEM((_K,), jnp.int32) for _ in range(3)]
             for _ in range(2)],
            [[pltpu.VMEM((_K,), jnp.float32) for _ in range(3)]
             for _ in range(2)],
            [[pltpu.VMEM((_K,), jnp.float32) for _ in range(3)]
             for _ in range(2)],
            [pltpu.VMEM((_K,), jnp.float32) for _ in range(2)],
            [pltpu.VMEM((3 * _K,), jnp.float32) for _ in range(2)],
            [pltpu.SemaphoreType.DMA for _ in range(2)],
            [pltpu.SemaphoreType.DMA for _ in range(2)],
            [pltpu.SemaphoreType.DMA for _ in range(2)],
        ],
    )
    def dist_kernel(xyzf_hbm, edge_hbm, cell_hbm,
                    dis_hbm, out2_hbm,
                    cell_v, idx2, off_s, off_d, s_comp, d_comp,
                    dis_v, out2_v, sem_idx, sem_g, sem_out):
        wid = lax.axis_index("s") * _NC + lax.axis_index("c")
        last = n_chunks - 1

        def cid(j):
            return jnp.minimum(wid + _NW * j, last)

        pltpu.sync_copy(cell_hbm, cell_v)
        cell_c = (cell_v[0], cell_v[1], cell_v[2])
        iota3 = lax.iota(jnp.int32, _L) * 3

        def fire_idx(p, c):
            pltpu.async_copy(edge_hbm.at[:, pl.ds(c * _K, _K)], idx2[p],
                             sem_idx[p])

        def wait_idx(p):
            pltpu.make_async_copy(edge_hbm.at[:, pl.ds(0, _K)], idx2[p],
                                  sem_idx[p]).wait()

        def scale_idx(p):
            def sg(g, carry):
                sl = pl.ds(g * _L, _L)
                v = idx2[p][0, sl] * 3
                off_s[p][0][sl] = v
                off_s[p][1][sl] = v + 1
                off_s[p][2][sl] = v + 2
                v = idx2[p][1, sl] * 3
                off_d[p][0][sl] = v
                off_d[p][1][sl] = v + 1
                off_d[p][2][sl] = v + 2
                return carry

            lax.fori_loop(0, _K // _L, sg, 0)

        def fire_gather(p):
            for k in range(3):
                pltpu.async_copy(xyzf_hbm.at[off_s[p][k]], s_comp[p][k],
                                 sem_g[p])
                pltpu.async_copy(xyzf_hbm.at[off_d[p][k]], d_comp[p][k],
                                 sem_g[p])

        def wait_gather(p):
            for k in range(3):
                pltpu.make_async_copy(xyzf_hbm.at[off_s[p][k]], s_comp[p][k],
                                      sem_g[p]).wait()
                pltpu.make_async_copy(xyzf_hbm.at[off_d[p][k]], d_comp[p][k],
                                      sem_g[p]).wait()

        def fire_out(p, c):
            pltpu.async_copy(dis_v[p], dis_hbm.at[pl.ds(c * _K, _K)],
                             sem_out[p])
            pltpu.async_copy(out2_v[p], out2_hbm.at[pl.ds(c * (3 * _K),
                                                          3 * _K)],
                             sem_out[p])

        def wait_out(p):
            pltpu.make_async_copy(dis_v[p], dis_hbm.at[pl.ds(0, _K)],
                                  sem_out[p]).wait()
            pltpu.make_async_copy(out2_v[p], out2_hbm.at[pl.ds(0, 3 * _K)],
                                  sem_out[p]).wait()

        def compute(p):
            def grp(g, carry2):
                e3 = g * (3 * _L) + iota3
                sumsq = jnp.zeros((_L,), jnp.float32)
                for comp in range(3):
                    s = s_comp[p][comp][pl.ds(g * _L, _L)]
                    d = d_comp[p][comp][pl.ds(g * _L, _L)]
                    dv = s - d
                    adv = jnp.abs(dv)
                    dv2 = jnp.minimum(cell_c[comp] - adv, adv)
                    mask2 = jnp.where(jnp.abs(dv2) == adv, 1.0, -1.0)
                    w = dv + 1e-8
                    mask = w / jnp.abs(w)
                    plsc.store_scatter(out2_v[p], [e3 + comp],
                                       dv2 * mask2 * mask)
                    sv = dv + 1e-9
                    sumsq = sumsq + sv * sv
                dis_v[p][pl.ds(g * _L, _L)] = sumsq * _newton_rsqrt(sumsq)
                return carry2

            lax.fori_loop(0, _K // _L, grp, 0)

        fire_idx(0, cid(0))
        fire_idx(1, cid(1))
        wait_idx(0)
        scale_idx(0)
        fire_gather(0)

        def body(t, carry):
            a = 2 * t
            b = 2 * t + 1
            wait_idx(1)
            scale_idx(1)
            fire_gather(1)
            wait_gather(0)

            @pl.when(t > 0)
            def _():
                wait_out(0)
            fire_idx(0, cid(a + 2))
            compute(0)
            fire_out(0, cid(a))
            wait_idx(0)
            scale_idx(0)
            fire_gather(0)
            wait_gather(1)

            @pl.when(t > 0)
            def _():
                wait_out(1)
            fire_idx(1, cid(b + 2))
            compute(1)
            fire_out(1, cid(b))
            return carry

        lax.fori_loop(0, n_iter // 2, body, 0)

        wait_idx(1)
        wait_gather(0)
        wait_out(0)
        wait_out(1)

    return dist_kernel


@jax.jit
def kernel(xyz, edge_index, cell):
    E = edge_index.shape[1]
    N = xyz.shape[0]
    xyzf = xyz.reshape(3 * N)
    cell16 = jnp.broadcast_to(cell.reshape(3, 1), (3, _L)).astype(jnp.float32)
    dis, out2f = _build(E)(xyzf, edge_index, cell16)
    return dis, out2f.reshape(E, 3)


# speedup vs baseline: 10.8565x; 1.1893x over previous
import functools

import jax
import jax.numpy as jnp
from jax import lax
from jax.experimental import pallas as pl
from jax.experimental.pallas import tpu as pltpu
from jax.experimental.pallas import tpu_sc as plsc

_NC = 2
_NS = 16
_NW = _NC * _NS
_L = 16

_K = 2000
_D = 8


def _newton_rsqrt(x):
    i = plsc.bitcast(x, jnp.int32)
    i = jnp.int32(0x5F3759DF) - (i >> 1)
    y = plsc.bitcast(i, jnp.float32)
    xh = x * 0.5
    for _ in range(2):
        y = y * (1.5 - xh * y * y)
    return y


def _build(E):
    per_w = E // _NW
    n_chunks = per_w // _K
    assert per_w % _K == 0 and _K % _L == 0 and n_chunks % 2 == 0
    mesh = plsc.VectorSubcoreMesh(core_axis_name="c", subcore_axis_name="s",
                                  num_cores=_NC, num_subcores=_NS)

    @functools.partial(
        pl.kernel,
        mesh=mesh,
        compiler_params=pltpu.CompilerParams(needs_layout_passes=False,
                                             use_tc_tiling_on_sc=False),
        out_type=(
            jax.ShapeDtypeStruct((E,), jnp.float32),
            jax.ShapeDtypeStruct((3 * E,), jnp.float32),
        ),
        scratch_types=[
            pltpu.VMEM((3, _L), jnp.float32),
            [pltpu.VMEM((_K,), jnp.int32) for _ in range(2)],
            [pltpu.VMEM((_K,), jnp.int32) for _ in range(2)],
            [pltpu.VMEM((_K, _D), jnp.float32) for _ in range(2)],
            [pltpu.VMEM((_K, _D), jnp.float32) for _ in range(2)],
            [pltpu.VMEM((_K,), jnp.float32) for _ in range(2)],
            [pltpu.VMEM((3 * _K,), jnp.float32) for _ in range(2)],
            [pltpu.SemaphoreType.DMA for _ in range(2)],
            [pltpu.SemaphoreType.DMA for _ in range(2)],
            [pltpu.SemaphoreType.DMA for _ in range(2)],
        ],
    )
    def dist_kernel(xyz8_hbm, ef_hbm, cell_hbm,
                    dis_hbm, out2_hbm,
                    cell_v, idx_s, idx_d, s_rows, d_rows, dis_v, out2_v,
                    sem_idx, sem_g, sem_out):
        wid = lax.axis_index("s") * _NC + lax.axis_index("c")
        base = wid * per_w

        pltpu.sync_copy(cell_hbm, cell_v)
        cell_c = (cell_v[0], cell_v[1], cell_v[2])
        iota = lax.iota(jnp.int32, _L)
        iota3 = iota * 3
        comp_i = tuple(jnp.full((_L,), k, jnp.int32) for k in range(3))

        def fire_idx(p, c):
            gb = base + c * _K
            pltpu.async_copy(ef_hbm.at[pl.ds(gb, _K)], idx_s[p], sem_idx[p])
            pltpu.async_copy(ef_hbm.at[pl.ds(E + gb, _K)], idx_d[p],
                             sem_idx[p])

        def wait_idx(p):
            pltpu.make_async_copy(ef_hbm.at[pl.ds(0, _K)], idx_s[p],
                                  sem_idx[p]).wait()
            pltpu.make_async_copy(ef_hbm.at[pl.ds(0, _K)], idx_d[p],
                                  sem_idx[p]).wait()

        def fire_gather(p):
            pltpu.async_copy(xyz8_hbm.at[idx_s[p]], s_rows[p], sem_g[p])
            pltpu.async_copy(xyz8_hbm.at[idx_d[p]], d_rows[p], sem_g[p])

        def wait_gather(p):
            pltpu.make_async_copy(xyz8_hbm.at[idx_s[p]], s_rows[p],
                                  sem_g[p]).wait()
            pltpu.make_async_copy(xyz8_hbm.at[idx_d[p]], d_rows[p],
                                  sem_g[p]).wait()

        def fire_out(p, c):
            gb = base + c * _K
            pltpu.async_copy(dis_v[p], dis_hbm.at[pl.ds(gb, _K)], sem_out[p])
            pltpu.async_copy(out2_v[p], out2_hbm.at[pl.ds(3 * gb, 3 * _K)],
                             sem_out[p])

        def wait_out(p):
            pltpu.make_async_copy(dis_v[p], dis_hbm.at[pl.ds(0, _K)],
                                  sem_out[p]).wait()
            pltpu.make_async_copy(out2_v[p], out2_hbm.at[pl.ds(0, 3 * _K)],
                                  sem_out[p]).wait()

        def compute(p):
            def grp(g, carry2):
                row = g * _L + iota
                e3 = g * (3 * _L) + iota3
                sumsq = jnp.zeros((_L,), jnp.float32)
                for comp in range(3):
                    s = plsc.load_gather(s_rows[p], [row, comp_i[comp]])
                    d = plsc.load_gather(d_rows[p], [row, comp_i[comp]])
                    dv = s - d
                    adv = jnp.abs(dv)
                    dv2 = jnp.minimum(cell_c[comp] - adv, adv)
                    sb = plsc.bitcast(dv + 1e-8, jnp.int32) & jnp.int32(
                        -2147483648)
                    sb = jnp.where(jnp.abs(dv2) == adv, sb,
                                   sb ^ jnp.int32(-2147483648))
                    out = plsc.bitcast(plsc.bitcast(dv2, jnp.int32) ^ sb,
                                       jnp.float32)
                    plsc.store_scatter(out2_v[p], [e3 + comp], out)
                    sv = dv + 1e-9
                    sumsq = sumsq + sv * sv
                dis_v[p][pl.ds(g * _L, _L)] = sumsq * _newton_rsqrt(sumsq)
                return carry2

            lax.fori_loop(0, _K // _L, grp, 0)

        last = n_chunks - 1

        fire_idx(0, 0)
        fire_idx(1, 1)
        wait_idx(0)
        fire_gather(0)

        def body(t, carry):
            a = 2 * t
            b = 2 * t + 1
            wait_idx(1)
            fire_gather(1)
            wait_gather(0)

            @pl.when(t > 0)
            def _():
                wait_out(0)
            fire_idx(0, jnp.minimum(a + 2, last))
            compute(0)
            fire_out(0, a)
            wait_idx(0)
            fire_gather(0)
            wait_gather(1)

            @pl.when(t > 0)
            def _():
                wait_out(1)
            fire_idx(1, jnp.minimum(b + 2, last))
            compute(1)
            fire_out(1, b)
            return carry

        lax.fori_loop(0, n_chunks // 2, body, 0)

        wait_idx(1)
        wait_gather(0)
        wait_out(0)
        wait_out(1)

    return dist_kernel


@jax.jit
def kernel(xyz, edge_index, cell):
    E = edge_index.shape[1]
    xyz8 = jnp.pad(xyz, ((0, 0), (0, _D - 3)))
    ef = edge_index.reshape(2 * E)
    cell16 = jnp.broadcast_to(cell.reshape(3, 1), (3, _L)).astype(jnp.float32)
    dis, out2f = _build(E)(xyz8, ef, cell16)
    return dis, out2f.reshape(E, 3)


# speedup vs baseline: 60.4409x; 5.5672x over previous
import functools

import jax
import jax.numpy as jnp
from jax import lax
from jax.experimental import pallas as pl
from jax.experimental.pallas import tpu as pltpu
from jax.experimental.pallas import tpu_sc as plsc

_NC = 2
_NS = 16
_NW = _NC * _NS
_L = 16

_K = 2000
_D = 8


def _newton_rsqrt(x):
    i = plsc.bitcast(x, jnp.int32)
    i = jnp.int32(0x5F3759DF) - (i >> 1)
    y = plsc.bitcast(i, jnp.float32)
    xh = x * 0.5
    for _ in range(2):
        y = y * (1.5 - xh * y * y)
    return y


_KD = 6400


def _build_detile(E):
    n_chunks = E // _KD
    assert E % _KD == 0
    n_iter = -(-n_chunks // _NW)
    if n_iter % 2:
        n_iter += 1
    mesh = plsc.VectorSubcoreMesh(core_axis_name="c", subcore_axis_name="s",
                                  num_cores=_NC, num_subcores=_NS)

    @functools.partial(
        pl.kernel,
        mesh=mesh,
        out_type=jax.ShapeDtypeStruct((2 * E,), jnp.int32),
        scratch_types=[
            [pltpu.VMEM((2, _KD), jnp.int32) for _ in range(2)],
            [pltpu.SemaphoreType.DMA for _ in range(2)],
            [pltpu.SemaphoreType.DMA for _ in range(2)],
        ],
    )
    def detile_kernel(edge_hbm, ef_hbm, bufs, sem_in, sem_out):
        wid = lax.axis_index("s") * _NC + lax.axis_index("c")
        last = n_chunks - 1

        def cid(j):
            return jnp.minimum(wid + _NW * j, last)

        def fire_in(p, c):
            pltpu.async_copy(edge_hbm.at[:, pl.ds(c * _KD, _KD)], bufs[p],
                             sem_in[p])

        def wait_in(p):
            pltpu.make_async_copy(edge_hbm.at[:, pl.ds(0, _KD)], bufs[p],
                                  sem_in[p]).wait()

        def fire_out(p, c):
            gb = c * _KD
            pltpu.async_copy(bufs[p].at[0], ef_hbm.at[pl.ds(gb, _KD)],
                             sem_out[p])
            pltpu.async_copy(bufs[p].at[1], ef_hbm.at[pl.ds(E + gb, _KD)],
                             sem_out[p])

        def wait_out(p):
            pltpu.make_async_copy(bufs[p].at[0], ef_hbm.at[pl.ds(0, _KD)],
                                  sem_out[p]).wait()
            pltpu.make_async_copy(bufs[p].at[1], ef_hbm.at[pl.ds(0, _KD)],
                                  sem_out[p]).wait()

        fire_in(0, cid(0))
        fire_in(1, cid(1))

        def body(t, carry):
            wait_in(0)
            fire_out(0, cid(2 * t))
            wait_in(1)
            fire_out(1, cid(2 * t + 1))
            wait_out(0)
            fire_in(0, cid(2 * t + 2))
            wait_out(1)
            fire_in(1, cid(2 * t + 3))
            return carry

        lax.fori_loop(0, n_iter // 2, body, 0)
        wait_in(0)
        wait_in(1)

    return detile_kernel


def _build(E):
    per_w = E // _NW
    n_chunks = per_w // _K
    assert per_w % _K == 0 and _K % _L == 0 and n_chunks % 2 == 0
    mesh = plsc.VectorSubcoreMesh(core_axis_name="c", subcore_axis_name="s",
                                  num_cores=_NC, num_subcores=_NS)

    @functools.partial(
        pl.kernel,
        mesh=mesh,
        compiler_params=pltpu.CompilerParams(needs_layout_passes=False,
                                             use_tc_tiling_on_sc=False),
        out_type=(
            jax.ShapeDtypeStruct((E,), jnp.float32),
            jax.ShapeDtypeStruct((E,), jnp.float32),
            jax.ShapeDtypeStruct((E,), jnp.float32),
            jax.ShapeDtypeStruct((E,), jnp.float32),
        ),
        scratch_types=[
            pltpu.VMEM((3, _L), jnp.float32),
            [pltpu.VMEM((_K,), jnp.int32) for _ in range(2)],
            [pltpu.VMEM((_K,), jnp.int32) for _ in range(2)],
            [pltpu.VMEM((_K, _D), jnp.float32) for _ in range(2)],
            [pltpu.VMEM((_K, _D), jnp.float32) for _ in range(2)],
            [pltpu.VMEM((_K,), jnp.float32) for _ in range(2)],
            [[pltpu.VMEM((_K,), jnp.float32) for _ in range(3)]
             for _ in range(2)],
            [pltpu.SemaphoreType.DMA for _ in range(2)],
            [pltpu.SemaphoreType.DMA for _ in range(2)],
            [pltpu.SemaphoreType.DMA for _ in range(2)],
        ],
    )
    def dist_kernel(xyz8_hbm, ef_hbm, cell_hbm,
                    dis_hbm, ox_hbm, oy_hbm, oz_hbm,
                    cell_v, idx_s, idx_d, s_rows, d_rows, dis_v, o_comp,
                    sem_idx, sem_g, sem_out):
        o_hbm = (ox_hbm, oy_hbm, oz_hbm)
        wid = lax.axis_index("s") * _NC + lax.axis_index("c")
        base = wid * per_w

        pltpu.sync_copy(cell_hbm, cell_v)
        cell_c = (cell_v[0], cell_v[1], cell_v[2])
        iota = lax.iota(jnp.int32, _L)
        comp_i = tuple(jnp.full((_L,), k, jnp.int32) for k in range(3))

        def fire_idx(p, c):
            gb = base + c * _K
            pltpu.async_copy(ef_hbm.at[pl.ds(gb, _K)], idx_s[p], sem_idx[p])
            pltpu.async_copy(ef_hbm.at[pl.ds(E + gb, _K)], idx_d[p],
                             sem_idx[p])

        def wait_idx(p):
            pltpu.make_async_copy(ef_hbm.at[pl.ds(0, _K)], idx_s[p],
                                  sem_idx[p]).wait()
            pltpu.make_async_copy(ef_hbm.at[pl.ds(0, _K)], idx_d[p],
                                  sem_idx[p]).wait()

        def fire_gather(p):
            pltpu.async_copy(xyz8_hbm.at[idx_s[p]], s_rows[p], sem_g[p])
            pltpu.async_copy(xyz8_hbm.at[idx_d[p]], d_rows[p], sem_g[p])

        def wait_gather(p):
            pltpu.make_async_copy(xyz8_hbm.at[idx_s[p]], s_rows[p],
                                  sem_g[p]).wait()
            pltpu.make_async_copy(xyz8_hbm.at[idx_d[p]], d_rows[p],
                                  sem_g[p]).wait()

        def fire_out(p, c):
            gb = base + c * _K
            pltpu.async_copy(dis_v[p], dis_hbm.at[pl.ds(gb, _K)], sem_out[p])
            for k in range(3):
                pltpu.async_copy(o_comp[p][k], o_hbm[k].at[pl.ds(gb, _K)],
                                 sem_out[p])

        def wait_out(p):
            pltpu.make_async_copy(dis_v[p], dis_hbm.at[pl.ds(0, _K)],
                                  sem_out[p]).wait()
            for k in range(3):
                pltpu.make_async_copy(o_comp[p][k],
                                      o_hbm[k].at[pl.ds(0, _K)],
                                      sem_out[p]).wait()

        def compute(p):
            def grp(g, carry2):
                row = g * _L + iota
                sumsq = jnp.zeros((_L,), jnp.float32)
                for comp in range(3):
                    s = plsc.load_gather(s_rows[p], [row, comp_i[comp]])
                    d = plsc.load_gather(d_rows[p], [row, comp_i[comp]])
                    dv = s - d
                    adv = jnp.abs(dv)
                    dv2 = jnp.minimum(cell_c[comp] - adv, adv)
                    sb = plsc.bitcast(dv + 1e-8, jnp.int32) & jnp.int32(
                        -2147483648)
                    sb = jnp.where(jnp.abs(dv2) == adv, sb,
                                   sb ^ jnp.int32(-2147483648))
                    out = plsc.bitcast(plsc.bitcast(dv2, jnp.int32) ^ sb,
                                       jnp.float32)
                    o_comp[p][comp][pl.ds(g * _L, _L)] = out
                    sv = dv + 1e-9
                    sumsq = sumsq + sv * sv
                dis_v[p][pl.ds(g * _L, _L)] = sumsq * _newton_rsqrt(sumsq)
                return carry2

            lax.fori_loop(0, _K // _L, grp, 0)

        last = n_chunks - 1

        fire_idx(0, 0)
        fire_idx(1, 1)
        wait_idx(0)
        fire_gather(0)

        def body(t, carry):
            a = 2 * t
            b = 2 * t + 1
            wait_idx(1)
            fire_gather(1)
            wait_gather(0)

            @pl.when(t > 0)
            def _():
                wait_out(0)
            fire_idx(0, jnp.minimum(a + 2, last))
            compute(0)
            fire_out(0, a)
            wait_idx(0)
            fire_gather(0)
            wait_gather(1)

            @pl.when(t > 0)
            def _():
                wait_out(1)
            fire_idx(1, jnp.minimum(b + 2, last))
            compute(1)
            fire_out(1, b)
            return carry

        lax.fori_loop(0, n_chunks // 2, body, 0)

        wait_idx(1)
        wait_gather(0)
        wait_out(0)
        wait_out(1)

    return dist_kernel


@jax.jit
def kernel(xyz, edge_index, cell):
    E = edge_index.shape[1]
    xyz8 = jnp.pad(xyz, ((0, 0), (0, _D - 3)))
    ef = _build_detile(E)(edge_index)
    cell16 = jnp.broadcast_to(cell.reshape(3, 1), (3, _L)).astype(jnp.float32)
    dis, ox, oy, oz = _build(E)(xyz8, ef, cell16)
    return dis, jnp.stack((ox, oy, oz), axis=-1)
